# baseline (speedup 1.0000x reference)
import functools

import jax
import jax.numpy as jnp
from jax import lax
from jax.experimental import pallas as pl
from jax.experimental.pallas import tpu as pltpu
from jax.experimental.pallas import tpu_sc as plsc

_D_MODEL = 4096
_NUM_EXPERTS = 64
_TOP_K = 8
_BLOCK_T = 1024
_N_TOKENS = 32768

_NW = 32
_ROWS_PER_W = _N_TOKENS // _NW
_CHUNK = 128


def _gate_tc_kernel(x_ref, w_ref, probs_ref, logits_ref, psum_ref):
    logits_t = jax.lax.dot_general(
        w_ref[...], x_ref[...],
        dimension_numbers=(((1,), (1,)), ((), ())),
        preferred_element_type=jnp.float32,
    )
    logits_ref[...] = logits_t.T
    m = jnp.max(logits_t, axis=0, keepdims=True)
    e = jnp.exp(logits_t - m)
    s = jnp.sum(e, axis=0, keepdims=True)
    probs_t = e / s
    probs_ref[...] = probs_t.T
    psum_ref[...] = jnp.sum(probs_t, axis=0, keepdims=True)


def _merge_top16(ak, av, bk, bv):
    bk_r = lax.rev(bk, (0,))
    bv_r = lax.rev(bv, (0,))
    ge = ak >= bk_r
    mk = jnp.where(ge, ak, bk_r)
    mv = jnp.where(ge, av, bv_r)
    return plsc.sort_key_val(mk, mv, descending=True)


def _sc_topk_body(probs_hbm, psum_hbm, outw_hbm, outi_hbm, buf, bps, ow, oi):
    c_idx = lax.axis_index("c")
    s_idx = lax.axis_index("s")
    wid = s_idx * 2 + c_idx
    base = wid * _ROWS_PER_W
    lane = lax.iota(jnp.int32, 16)
    mask8 = lane < _TOP_K

    def do_chunk(c, carry0):
        t0 = base + c * _CHUNK
        pltpu.sync_copy(
            probs_hbm.at[pl.ds(t0 * _NUM_EXPERTS, _CHUNK * _NUM_EXPERTS)],
            buf,
        )
        pltpu.sync_copy(psum_hbm.at[pl.ds(t0, _CHUNK)],
                        bps.at[pl.ds(0, _CHUNK)])

        def do_token(t, carry1):
            b = t * _NUM_EXPERTS
            vs = [buf[pl.ds(b + 16 * j, 16)] for j in range(4)]
            ids = [lane + 16 * j for j in range(4)]
            total = bps[pl.ds(t, 16)][0]
            srt = [
                plsc.sort_key_val(vs[j], ids[j], descending=True)
                for j in range(4)
            ]
            k01, v01 = _merge_top16(*srt[0], *srt[1])
            k23, v23 = _merge_top16(*srt[2], *srt[3])
            kk, vv = _merge_top16(k01, v01, k23, v23)
            plsc.store_compressed(
                ow.at[pl.ds(t * _TOP_K, 16)], kk / total, mask=mask8
            )
            plsc.store_compressed(
                oi.at[pl.ds(t * _TOP_K, 16)], vv, mask=mask8
            )
            return carry1

        lax.fori_loop(0, _CHUNK, do_token, 0)
        pltpu.sync_copy(
            ow.at[pl.ds(0, _CHUNK * _TOP_K)],
            outw_hbm.at[pl.ds(t0 * _TOP_K, _CHUNK * _TOP_K)],
        )
        pltpu.sync_copy(
            oi.at[pl.ds(0, _CHUNK * _TOP_K)],
            outi_hbm.at[pl.ds(t0 * _TOP_K, _CHUNK * _TOP_K)],
        )
        return carry0

    lax.fori_loop(0, _ROWS_PER_W // _CHUNK, do_chunk, 0)


_sc_topk = functools.partial(
    pl.kernel,
    out_type=[
        jax.ShapeDtypeStruct((_N_TOKENS * _TOP_K,), jnp.float32),
        jax.ShapeDtypeStruct((_N_TOKENS * _TOP_K,), jnp.int32),
    ],
    mesh=plsc.VectorSubcoreMesh(core_axis_name="c", subcore_axis_name="s"),
    compiler_params=pltpu.CompilerParams(needs_layout_passes=False),
    scratch_types=[
        pltpu.VMEM((_CHUNK * _NUM_EXPERTS,), jnp.float32),
        pltpu.VMEM((_CHUNK + 16,), jnp.float32),
        pltpu.VMEM((_CHUNK * _TOP_K + 16,), jnp.float32),
        pltpu.VMEM((_CHUNK * _TOP_K + 16,), jnp.int32),
    ],
)(_sc_topk_body)


@functools.partial(jax.jit, static_argnames=())
def kernel(x, W):
    n_tokens, d_model = x.shape
    n_experts = W.shape[0]
    grid = (n_tokens // _BLOCK_T,)
    probs, logits, psum = pl.pallas_call(
        _gate_tc_kernel,
        grid=grid,
        in_specs=[
            pl.BlockSpec((_BLOCK_T, d_model), lambda i: (i, 0)),
            pl.BlockSpec((n_experts, d_model), lambda i: (0, 0)),
        ],
        out_specs=[
            pl.BlockSpec((_BLOCK_T, n_experts), lambda i: (i, 0)),
            pl.BlockSpec((_BLOCK_T, n_experts), lambda i: (i, 0)),
            pl.BlockSpec((1, _BLOCK_T), lambda i: (0, i)),
        ],
        out_shape=[
            jax.ShapeDtypeStruct((n_tokens, n_experts), jnp.float32),
            jax.ShapeDtypeStruct((n_tokens, n_experts), jnp.float32),
            jax.ShapeDtypeStruct((1, n_tokens), jnp.float32),
        ],
        compiler_params=pltpu.CompilerParams(
            dimension_semantics=("arbitrary",),
        ),
    )(x, W)
    topk_w_flat, topk_i_flat = _sc_topk(probs.reshape(-1), psum.reshape(-1))
    topk_w = topk_w_flat.reshape(n_tokens, _TOP_K)
    topk_i = topk_i_flat.reshape(n_tokens, _TOP_K)
    return (topk_w, probs, topk_i, logits)
